# baseline (speedup 1.0000x reference)
import functools

import jax
import jax.numpy as jnp
from jax import lax
from jax.experimental import pallas as pl
from jax.experimental.pallas import tpu as pltpu
from jax.experimental.pallas import tpu_sc as plsc

_ROWS = 16384
_SEQ = 200
_D = 64
_B = _ROWS * _SEQ
_NW = 32
_BPW = _B // _NW
_C = 1024
_NCHUNK = _BPW // _C

_mesh = plsc.VectorSubcoreMesh(core_axis_name="c", subcore_axis_name="s")


@functools.partial(
    pl.kernel,
    mesh=_mesh,
    out_type=jax.ShapeDtypeStruct((_B, _D), jnp.float32),
    scratch_types=[
        pltpu.VMEM((_C,), jnp.int32),
        pltpu.VMEM((_C, _D), jnp.float32),
        pltpu.SemaphoreType.DMA,
    ],
    compiler_params=pltpu.CompilerParams(use_tc_tiling_on_sc=False),
)
def _embed_sc(idx_hbm, table_hbm, out_hbm, idx_v, rows_v, sem):
    wid = lax.axis_index("s") * 2 + lax.axis_index("c")
    base = wid * _BPW

    def body(i, carry):
        off = base + i * _C
        pltpu.sync_copy(idx_hbm.at[pl.ds(off, _C)], idx_v)
        pltpu.async_copy(table_hbm.at[idx_v], rows_v, sem).wait()
        pltpu.sync_copy(rows_v, out_hbm.at[pl.ds(off, _C)])
        return carry

    lax.fori_loop(0, _NCHUNK, body, 0)


def kernel(input, weight):
    idx = jnp.asarray(input, jnp.int32).reshape(_B)
    out = _embed_sc(idx, weight)
    return out.reshape(_ROWS, _SEQ, _D)


# speedup vs baseline: 1.0339x; 1.0339x over previous
import functools

import jax
import jax.numpy as jnp
from jax import lax
from jax.experimental import pallas as pl
from jax.experimental.pallas import tpu as pltpu
from jax.experimental.pallas import tpu_sc as plsc

_ROWS = 16384
_SEQ = 200
_D = 64
_B = _ROWS * _SEQ
_NW = 32
_BPW = _B // _NW
_C = 800
_NCHUNK = _BPW // _C

_mesh = plsc.VectorSubcoreMesh(core_axis_name="c", subcore_axis_name="s")


@functools.partial(
    pl.kernel,
    mesh=_mesh,
    out_type=jax.ShapeDtypeStruct((_B, _D), jnp.float32),
    scratch_types=[
        pltpu.VMEM((_C,), jnp.int32),
        pltpu.VMEM((_C,), jnp.int32),
        pltpu.VMEM((_C, _D), jnp.float32),
        pltpu.VMEM((_C, _D), jnp.float32),
        pltpu.SemaphoreType.DMA,
        pltpu.SemaphoreType.DMA,
        pltpu.SemaphoreType.DMA,
        pltpu.SemaphoreType.DMA,
        pltpu.SemaphoreType.DMA,
        pltpu.SemaphoreType.DMA,
    ],
    compiler_params=pltpu.CompilerParams(use_tc_tiling_on_sc=False),
)
def _embed_sc(idx_hbm, table_hbm, out_hbm, idx_v0, idx_v1, rows_v0, rows_v1,
              si0, si1, sg0, sg1, so0, so1):
    wid = lax.axis_index("s") * 2 + lax.axis_index("c")
    base = wid * _BPW

    idx_v = (idx_v0, idx_v1)
    rows_v = (rows_v0, rows_v1)
    sem_i = (si0, si1)
    sem_g = (sg0, sg1)
    sem_o = (so0, so1)

    def idx_desc(i, b):
        return pltpu.make_async_copy(
            idx_hbm.at[pl.ds(base + i * _C, _C)], idx_v[b], sem_i[b])

    def gather_desc(b):
        return pltpu.make_async_copy(
            table_hbm.at[idx_v[b]], rows_v[b], sem_g[b])

    def scatter_desc(i, b):
        return pltpu.make_async_copy(
            rows_v[b], out_hbm.at[pl.ds(base + i * _C, _C)], sem_o[b])

    d = idx_desc(0, 0)
    d.start()
    d.wait()
    idx_desc(1, 1).start()
    gather_desc(0).start()

    @pl.loop(0, _NCHUNK, step=2)
    def _pair(i):
        for b in (0, 1):
            chunk = i + b
            nb = 1 - b

            @pl.when(chunk + 1 < _NCHUNK)
            def _():
                idx_desc(0, nb).wait()
                @pl.when(chunk >= 1)
                def _():
                    scatter_desc(0, nb).wait()
                gather_desc(nb).start()

            gather_desc(b).wait()
            @pl.when(chunk + 2 < _NCHUNK)
            def _():
                idx_desc(chunk + 2, b).start()
            scatter_desc(chunk, b).start()

    scatter_desc(0, 0).wait()
    scatter_desc(0, 1).wait()


def kernel(input, weight):
    idx = jnp.asarray(input, jnp.int32).reshape(_B)
    out = _embed_sc(idx, weight)
    return out.reshape(_ROWS, _SEQ, _D)


# speedup vs baseline: 1.0341x; 1.0002x over previous
import functools

import jax
import jax.numpy as jnp
from jax import lax
from jax.experimental import pallas as pl
from jax.experimental.pallas import tpu as pltpu
from jax.experimental.pallas import tpu_sc as plsc

_ROWS = 16384
_SEQ = 200
_D = 64
_NW = 32
_RPW = _ROWS // _NW
_R = 4
_C = _R * _SEQ
_NCHUNK = _RPW // _R

_mesh = plsc.VectorSubcoreMesh(core_axis_name="c", subcore_axis_name="s")


@functools.partial(
    pl.kernel,
    mesh=_mesh,
    out_type=jax.ShapeDtypeStruct((_ROWS, _SEQ, _D), jnp.float32),
    scratch_types=[
        pltpu.VMEM((_C,), jnp.int32),
        pltpu.VMEM((_C,), jnp.int32),
        pltpu.VMEM((_C, _D), jnp.float32),
        pltpu.VMEM((_C, _D), jnp.float32),
        pltpu.SemaphoreType.DMA,
        pltpu.SemaphoreType.DMA,
        pltpu.SemaphoreType.DMA,
        pltpu.SemaphoreType.DMA,
        pltpu.SemaphoreType.DMA,
        pltpu.SemaphoreType.DMA,
    ],
    compiler_params=pltpu.CompilerParams(use_tc_tiling_on_sc=False),
)
def _embed_sc(idx_hbm, table_hbm, out_hbm, idx_v0, idx_v1, rows_v0, rows_v1,
              si0, si1, sg0, sg1, so0, so1):
    wid = lax.axis_index("s") * 2 + lax.axis_index("c")
    base = wid * _RPW

    idx_v = (idx_v0, idx_v1)
    rows_v = (rows_v0, rows_v1)
    sem_i = (si0, si1)
    sem_g = (sg0, sg1)
    sem_o = (so0, so1)

    def idx_start(i, b):
        row0 = base + i * _R
        for k in range(_R):
            pltpu.make_async_copy(
                idx_hbm.at[row0 + k, :],
                idx_v[b].at[pl.ds(k * _SEQ, _SEQ)], sem_i[b]).start()

    def idx_wait(b):
        for k in range(_R):
            pltpu.make_async_copy(
                idx_hbm.at[0, :],
                idx_v[b].at[pl.ds(k * _SEQ, _SEQ)], sem_i[b]).wait()

    def gather_desc(b):
        return pltpu.make_async_copy(
            table_hbm.at[idx_v[b]], rows_v[b], sem_g[b])

    def scatter_start(i, b):
        row0 = base + i * _R
        for k in range(_R):
            pltpu.make_async_copy(
                rows_v[b].at[pl.ds(k * _SEQ, _SEQ), :],
                out_hbm.at[row0 + k], sem_o[b]).start()

    def scatter_wait(b):
        for k in range(_R):
            pltpu.make_async_copy(
                rows_v[b].at[pl.ds(k * _SEQ, _SEQ), :],
                out_hbm.at[0], sem_o[b]).wait()

    idx_start(0, 0)
    idx_wait(0)
    idx_start(1, 1)
    gather_desc(0).start()

    @pl.loop(0, _NCHUNK, step=2)
    def _pair(i):
        for b in (0, 1):
            chunk = i + b
            nb = 1 - b

            @pl.when(chunk + 1 < _NCHUNK)
            def _():
                idx_wait(nb)
                @pl.when(chunk >= 1)
                def _():
                    scatter_wait(nb)
                gather_desc(nb).start()

            gather_desc(b).wait()
            @pl.when(chunk + 2 < _NCHUNK)
            def _():
                idx_start(chunk + 2, b)
            scatter_start(chunk, b)

    scatter_wait(0)
    scatter_wait(1)


def kernel(input, weight):
    return _embed_sc(jnp.asarray(input, jnp.int32), weight)
